# Initial kernel scaffold; baseline (speedup 1.0000x reference)
import functools

import jax
import jax.numpy as jnp
from jax import lax
from jax.experimental import pallas as pl
from jax.experimental.pallas import tpu as pltpu
from jax.experimental.pallas import tpu_sc as plsc

_NC = 2
_NS = 16
_LANES = 16


def _sc_gather(B, V, EMB, D, NNEG):
    nw = _NC * _NS
    assert B % nw == 0
    bpw = B // nw
    assert bpw <= 128
    n_full = EMB // _LANES
    tail = D - n_full * _LANES
    tail_off = D - _LANES

    mesh = plsc.VectorSubcoreMesh(core_axis_name="c", subcore_axis_name="s")

    @functools.partial(
        pl.kernel,
        out_type=(
            jax.ShapeDtypeStruct((B, EMB), jnp.float32),
            jax.ShapeDtypeStruct((B, D), jnp.float32),
            jax.ShapeDtypeStruct((B, D), jnp.float32),
        ),
        mesh=mesh,
        scratch_types=(
            pltpu.VMEM((bpw,), jnp.int32),
            pltpu.VMEM((bpw,), jnp.int32),
            pltpu.VMEM((NNEG, bpw), jnp.int32),
            pltpu.VMEM((bpw, EMB), jnp.float32),
            pltpu.VMEM((bpw, D), jnp.float32),
            pltpu.VMEM((bpw, D), jnp.float32),
            pltpu.SemaphoreType.DMA,
        ),
    )
    def sc_fn(tgt_hbm, ctx_hbm, negt_hbm, iemb_hbm, oemb_hbm,
              v_out, u_out, us_out,
              tidx_v, cidx_v, nidx_v, vbuf, rowbuf, accbuf, sem):
        wid = lax.axis_index("s") * _NC + lax.axis_index("c")
        base = wid * bpw

        pltpu.sync_copy(tgt_hbm.at[pl.ds(base, bpw)], tidx_v)
        pltpu.async_copy(iemb_hbm.at[tidx_v], vbuf, sem).wait()
        pltpu.sync_copy(vbuf, v_out.at[pl.ds(base, bpw)])

        pltpu.sync_copy(ctx_hbm.at[pl.ds(base, bpw)], cidx_v)
        pltpu.async_copy(oemb_hbm.at[cidx_v], rowbuf, sem).wait()
        pltpu.sync_copy(rowbuf, u_out.at[pl.ds(base, bpw)])

        pltpu.sync_copy(negt_hbm.at[:, pl.ds(base, bpw)], nidx_v)
        pltpu.async_copy(oemb_hbm.at[nidx_v.at[0]], accbuf, sem).wait()

        lanes = lax.iota(jnp.int32, _LANES)
        tail_keep = lanes >= (_LANES - tail)

        def n_body(n, carry):
            pltpu.async_copy(oemb_hbm.at[nidx_v.at[n]], rowbuf, sem).wait()

            def r_body(r, c2):
                for s in range(n_full):
                    plsc.addupdate(
                        accbuf.at[r, pl.ds(s * _LANES, _LANES)],
                        rowbuf[r, pl.ds(s * _LANES, _LANES)],
                    )
                xt = rowbuf[r, pl.ds(tail_off, _LANES)]
                xt = jnp.where(tail_keep, xt, 0.0)
                plsc.addupdate(accbuf.at[r, pl.ds(tail_off, _LANES)], xt)
                return c2

            return lax.fori_loop(0, bpw, r_body, carry)

        lax.fori_loop(1, NNEG, n_body, 0)
        pltpu.sync_copy(accbuf, us_out.at[pl.ds(base, bpw)])

    return sc_fn


def _tc_tail(B, EMB, TNUM, D):
    def body(v_ref, u_ref, us_ref, ty_ref, tm_ref, wt_ref, loss_ref, tloss_ref):
        v = v_ref[...]
        tp = jnp.dot(v, wt_ref[...], preferred_element_type=jnp.float32)
        ty = ty_ref[...]
        tm = tm_ref[...]
        bce = tm * (jnp.maximum(tp, 0.0) - tp * ty
                    + jnp.log(1.0 + jnp.exp(-jnp.abs(tp))))
        tloss_ref[0, 0] = jnp.sum(bce) / (B * TNUM)

        sig = 1.0 / (1.0 + jnp.exp(-tp))
        u = u_ref[...]
        us = us_ref[...]
        pos = (jnp.sum(u[:, :EMB] * v, axis=1, keepdims=True)
               + jnp.sum(u[:, EMB:] * sig, axis=1, keepdims=True))
        neg = (jnp.sum(us[:, :EMB] * v, axis=1, keepdims=True)
               + jnp.sum(us[:, EMB:] * sig, axis=1, keepdims=True))

        def logsig(x):
            return jnp.minimum(x, 0.0) - jnp.log(1.0 + jnp.exp(-jnp.abs(x)))

        loss_vec = logsig(pos) + logsig(-neg)
        loss_ref[0, 0] = -(jnp.sum(loss_vec) / B)

    return pl.pallas_call(
        body,
        out_shape=(
            jax.ShapeDtypeStruct((1, 1), jnp.float32),
            jax.ShapeDtypeStruct((1, 1), jnp.float32),
        ),
        out_specs=(
            pl.BlockSpec(memory_space=pltpu.SMEM),
            pl.BlockSpec(memory_space=pltpu.SMEM),
        ),
    )


def kernel(target_input, type_input, context, types, neg, type_mask,
           input_emb, output_emb, type_W):
    del type_input
    B = target_input.shape[0]
    EMB = input_emb.shape[1]
    V, D = output_emb.shape
    TNUM = type_W.shape[0]
    NNEG = neg.shape[1]

    neg_t = jnp.transpose(neg)
    sc = _sc_gather(B, V, EMB, D, NNEG)
    v, u, usum = sc(target_input, context, neg_t, input_emb, output_emb)

    loss, tloss = _tc_tail(B, EMB, TNUM, D)(
        v, u, usum, types, type_mask, jnp.transpose(type_W))
    return (loss[0, 0], tloss[0, 0])


# speedup vs baseline: 1.9189x; 1.9189x over previous
import functools

import jax
import jax.numpy as jnp
from jax import lax
from jax.experimental import pallas as pl
from jax.experimental.pallas import tpu as pltpu
from jax.experimental.pallas import tpu_sc as plsc

_NC = 2
_NS = 16
_LANES = 16


def _sc_gather(B, V, EMB, DP, NNEG):
    nw = _NC * _NS
    assert B % nw == 0
    bpw = B // nw
    assert bpw <= 128
    assert DP % _LANES == 0 and EMB % _LANES == 0
    n_slices = DP // _LANES

    mesh = plsc.VectorSubcoreMesh(core_axis_name="c", subcore_axis_name="s")

    @functools.partial(
        pl.kernel,
        out_type=(
            jax.ShapeDtypeStruct((B, EMB), jnp.float32),
            jax.ShapeDtypeStruct((B, DP), jnp.float32),
            jax.ShapeDtypeStruct((B, DP), jnp.float32),
        ),
        mesh=mesh,
        compiler_params=pltpu.CompilerParams(use_tc_tiling_on_sc=False),
        scratch_types=(
            pltpu.VMEM((bpw,), jnp.int32),
            pltpu.VMEM((bpw,), jnp.int32),
            pltpu.VMEM((NNEG, bpw), jnp.int32),
            pltpu.VMEM((bpw, EMB), jnp.float32),
            pltpu.VMEM((bpw, DP), jnp.float32),
            pltpu.VMEM((bpw, DP), jnp.float32),
            pltpu.SemaphoreType.DMA,
        ),
    )
    def sc_fn(tgt_hbm, ctx_hbm, negw_hbm, iemb_hbm, oemb_hbm,
              v_out, u_out, us_out,
              tidx_v, cidx_v, nidx_v, vbuf, rowbuf, accbuf, sem):
        wid = lax.axis_index("s") * _NC + lax.axis_index("c")
        base = wid * bpw

        pltpu.sync_copy(tgt_hbm.at[pl.ds(base, bpw)], tidx_v)
        pltpu.async_copy(iemb_hbm.at[tidx_v], vbuf, sem).wait()
        pltpu.sync_copy(vbuf, v_out.at[pl.ds(base, bpw)])

        pltpu.sync_copy(ctx_hbm.at[pl.ds(base, bpw)], cidx_v)
        pltpu.async_copy(oemb_hbm.at[cidx_v], rowbuf, sem).wait()
        pltpu.sync_copy(rowbuf, u_out.at[pl.ds(base, bpw)])

        pltpu.sync_copy(negw_hbm.at[wid], nidx_v)
        pltpu.async_copy(oemb_hbm.at[nidx_v.at[0]], accbuf, sem).wait()

        def n_body(n, carry):
            pltpu.async_copy(oemb_hbm.at[nidx_v.at[n]], rowbuf, sem).wait()

            def r_body(r, c2):
                for s in range(n_slices):
                    plsc.addupdate(
                        accbuf.at[r, pl.ds(s * _LANES, _LANES)],
                        rowbuf[r, pl.ds(s * _LANES, _LANES)],
                    )
                return c2

            return lax.fori_loop(0, bpw, r_body, carry)

        lax.fori_loop(1, NNEG, n_body, 0)
        pltpu.sync_copy(accbuf, us_out.at[pl.ds(base, bpw)])

    return sc_fn


def _tc_tail(B, EMB, TNUM, D):
    def body(v_ref, u_ref, us_ref, ty_ref, tm_ref, wt_ref, loss_ref, tloss_ref):
        v = v_ref[...]
        tp = jnp.dot(v, wt_ref[...], preferred_element_type=jnp.float32)
        ty = ty_ref[...]
        tm = tm_ref[...]
        bce = tm * (jnp.maximum(tp, 0.0) - tp * ty
                    + jnp.log(1.0 + jnp.exp(-jnp.abs(tp))))
        tloss_ref[0, 0] = jnp.sum(bce) / (B * TNUM)

        sig = 1.0 / (1.0 + jnp.exp(-tp))
        u = u_ref[...]
        us = us_ref[...]
        pos = (jnp.sum(u[:, :EMB] * v, axis=1, keepdims=True)
               + jnp.sum(u[:, EMB:D] * sig, axis=1, keepdims=True))
        neg = (jnp.sum(us[:, :EMB] * v, axis=1, keepdims=True)
               + jnp.sum(us[:, EMB:D] * sig, axis=1, keepdims=True))

        def logsig(x):
            return jnp.minimum(x, 0.0) - jnp.log(1.0 + jnp.exp(-jnp.abs(x)))

        loss_vec = logsig(pos) + logsig(-neg)
        loss_ref[0, 0] = -(jnp.sum(loss_vec) / B)

    return pl.pallas_call(
        body,
        out_shape=(
            jax.ShapeDtypeStruct((1, 1), jnp.float32),
            jax.ShapeDtypeStruct((1, 1), jnp.float32),
        ),
        out_specs=(
            pl.BlockSpec(memory_space=pltpu.SMEM),
            pl.BlockSpec(memory_space=pltpu.SMEM),
        ),
    )


def kernel(target_input, type_input, context, types, neg, type_mask,
           input_emb, output_emb, type_W):
    del type_input
    B = target_input.shape[0]
    EMB = input_emb.shape[1]
    V, D = output_emb.shape
    TNUM = type_W.shape[0]
    NNEG = neg.shape[1]

    DP = (D + _LANES - 1) // _LANES * _LANES
    oemb_p = jnp.pad(output_emb, ((0, 0), (0, DP - D)))

    nw = _NC * _NS
    bpw = B // nw
    neg_w = jnp.transpose(neg.reshape(nw, bpw, NNEG), (0, 2, 1))
    sc = _sc_gather(B, V, EMB, DP, NNEG)
    v, u, usum = sc(target_input, context, neg_w, input_emb, oemb_p)

    loss, tloss = _tc_tail(B, EMB, TNUM, D)(
        v, u, usum, types, type_mask, jnp.transpose(type_W))
    return (loss[0, 0], tloss[0, 0])


# speedup vs baseline: 4.0158x; 2.0928x over previous
import functools

import jax
import jax.numpy as jnp
from jax import lax
from jax.experimental import pallas as pl
from jax.experimental.pallas import tpu as pltpu
from jax.experimental.pallas import tpu_sc as plsc

_NC = 2
_NS = 16
_LANES = 16


def _sc_gather(B, V, EMB, DP, NNEG):
    nw = _NC * _NS
    assert B % nw == 0
    bpw = B // nw
    assert bpw <= 128
    assert DP % _LANES == 0 and EMB % _LANES == 0
    n_slices = DP // _LANES

    mesh = plsc.VectorSubcoreMesh(core_axis_name="c", subcore_axis_name="s")

    @functools.partial(
        pl.kernel,
        out_type=(
            jax.ShapeDtypeStruct((B, EMB), jnp.float32),
            jax.ShapeDtypeStruct((B, DP), jnp.float32),
            jax.ShapeDtypeStruct((B, DP), jnp.float32),
        ),
        mesh=mesh,
        compiler_params=pltpu.CompilerParams(use_tc_tiling_on_sc=False),
        scratch_types=(
            pltpu.VMEM((bpw,), jnp.int32),
            pltpu.VMEM((bpw,), jnp.int32),
            pltpu.VMEM((NNEG, bpw), jnp.int32),
            pltpu.VMEM((bpw, EMB), jnp.float32),
            pltpu.VMEM((bpw, DP), jnp.float32),
            pltpu.VMEM((bpw, DP), jnp.float32),
            pltpu.SemaphoreType.DMA,
        ),
    )
    def sc_fn(tgt_hbm, ctx_hbm, negw_hbm, iemb_hbm, oemb_hbm,
              v_out, u_out, us_out,
              tidx_v, cidx_v, nidx_v, vbuf, rowbuf, accbuf, sem):
        wid = lax.axis_index("s") * _NC + lax.axis_index("c")
        base = wid * bpw

        pltpu.sync_copy(tgt_hbm.at[pl.ds(base, bpw)], tidx_v)
        pltpu.async_copy(iemb_hbm.at[tidx_v], vbuf, sem).wait()
        pltpu.sync_copy(vbuf, v_out.at[pl.ds(base, bpw)])

        pltpu.sync_copy(ctx_hbm.at[pl.ds(base, bpw)], cidx_v)
        pltpu.async_copy(oemb_hbm.at[cidx_v], rowbuf, sem).wait()
        pltpu.sync_copy(rowbuf, u_out.at[pl.ds(base, bpw)])

        pltpu.sync_copy(negw_hbm.at[wid], nidx_v)
        pltpu.async_copy(oemb_hbm.at[nidx_v.at[0]], accbuf, sem).wait()

        def n_body(n, carry):
            pltpu.async_copy(oemb_hbm.at[nidx_v.at[n]], rowbuf, sem).wait()

            def r_body(r, c2):
                for s in range(n_slices):
                    plsc.addupdate(
                        accbuf.at[r, pl.ds(s * _LANES, _LANES)],
                        rowbuf[r, pl.ds(s * _LANES, _LANES)],
                    )
                return c2

            return lax.fori_loop(0, bpw, r_body, carry)

        lax.fori_loop(1, NNEG, n_body, 0)
        pltpu.sync_copy(accbuf, us_out.at[pl.ds(base, bpw)])

    return sc_fn


def _tc_pad(V, D, DP, rows):
    assert V % rows == 0 and rows % 8 == 0

    def body(i_ref, o_ref):
        o_ref[:, :D] = i_ref[...]

    return pl.pallas_call(
        body,
        grid=(V // rows,),
        in_specs=[pl.BlockSpec((rows, D), lambda i: (i, 0))],
        out_specs=pl.BlockSpec((rows, DP), lambda i: (i, 0)),
        out_shape=jax.ShapeDtypeStruct((V, DP), jnp.float32),
    )


def _tc_tail(B, EMB, TNUM, D):
    def body(v_ref, u_ref, us_ref, ty_ref, tm_ref, wt_ref, loss_ref, tloss_ref):
        v = v_ref[...]
        tp = jnp.dot(v, wt_ref[...], preferred_element_type=jnp.float32)
        ty = ty_ref[...]
        tm = tm_ref[...]
        bce = tm * (jnp.maximum(tp, 0.0) - tp * ty
                    + jnp.log(1.0 + jnp.exp(-jnp.abs(tp))))
        tloss_ref[0, 0] = jnp.sum(bce) / (B * TNUM)

        sig = 1.0 / (1.0 + jnp.exp(-tp))
        u = u_ref[...]
        us = us_ref[...]
        pos = (jnp.sum(u[:, :EMB] * v, axis=1, keepdims=True)
               + jnp.sum(u[:, EMB:D] * sig, axis=1, keepdims=True))
        neg = (jnp.sum(us[:, :EMB] * v, axis=1, keepdims=True)
               + jnp.sum(us[:, EMB:D] * sig, axis=1, keepdims=True))

        def logsig(x):
            return jnp.minimum(x, 0.0) - jnp.log(1.0 + jnp.exp(-jnp.abs(x)))

        loss_vec = logsig(pos) + logsig(-neg)
        loss_ref[0, 0] = -(jnp.sum(loss_vec) / B)

    return pl.pallas_call(
        body,
        out_shape=(
            jax.ShapeDtypeStruct((1, 1), jnp.float32),
            jax.ShapeDtypeStruct((1, 1), jnp.float32),
        ),
        out_specs=(
            pl.BlockSpec(memory_space=pltpu.SMEM),
            pl.BlockSpec(memory_space=pltpu.SMEM),
        ),
    )


def kernel(target_input, type_input, context, types, neg, type_mask,
           input_emb, output_emb, type_W):
    del type_input
    B = target_input.shape[0]
    EMB = input_emb.shape[1]
    V, D = output_emb.shape
    TNUM = type_W.shape[0]
    NNEG = neg.shape[1]

    DP = (D + _LANES - 1) // _LANES * _LANES
    oemb_p = _tc_pad(V, D, DP, rows=5000)(output_emb)

    nw = _NC * _NS
    bpw = B // nw
    neg_w = jnp.transpose(neg.reshape(nw, bpw, NNEG), (0, 2, 1))
    sc = _sc_gather(B, V, EMB, DP, NNEG)
    v, u, usum = sc(target_input, context, neg_w, input_emb, oemb_p)

    loss, tloss = _tc_tail(B, EMB, TNUM, D)(
        v, u, usum, types, type_mask, jnp.transpose(type_W))
    return (loss[0, 0], tloss[0, 0])


# speedup vs baseline: 4.1966x; 1.0450x over previous
import functools

import jax
import jax.numpy as jnp
from jax import lax
from jax.experimental import pallas as pl
from jax.experimental.pallas import tpu as pltpu
from jax.experimental.pallas import tpu_sc as plsc

_NC = 2
_NS = 16
_LANES = 16


def _sc_gather(B, V, EMB, DP, NNEG):
    nw = _NC * _NS
    assert B % nw == 0
    bpw = B // nw
    assert bpw <= 128
    assert DP % _LANES == 0 and EMB % _LANES == 0
    n_slices = DP // _LANES

    mesh = plsc.VectorSubcoreMesh(core_axis_name="c", subcore_axis_name="s")

    @functools.partial(
        pl.kernel,
        out_type=(
            jax.ShapeDtypeStruct((B, EMB), jnp.float32),
            jax.ShapeDtypeStruct((B, DP), jnp.float32),
            jax.ShapeDtypeStruct((B, DP), jnp.float32),
        ),
        mesh=mesh,
        compiler_params=pltpu.CompilerParams(use_tc_tiling_on_sc=False),
        scratch_types=(
            pltpu.VMEM((bpw,), jnp.int32),
            pltpu.VMEM((bpw,), jnp.int32),
            pltpu.VMEM((NNEG, bpw), jnp.int32),
            pltpu.VMEM((bpw, EMB), jnp.float32),
            pltpu.VMEM((bpw, DP), jnp.float32),
            pltpu.VMEM((bpw, DP), jnp.float32),
            pltpu.VMEM((bpw, DP), jnp.float32),
            pltpu.VMEM((bpw, DP), jnp.float32),
            pltpu.SemaphoreType.DMA,
            pltpu.SemaphoreType.DMA,
            pltpu.SemaphoreType.DMA,
            pltpu.SemaphoreType.DMA,
            pltpu.SemaphoreType.DMA,
        ),
    )
    def sc_fn(tgt_hbm, ctx_hbm, negw_hbm, iemb_hbm, oemb_hbm,
              v_out, u_out, us_out,
              tidx_v, cidx_v, nidx_v, vbuf, ubuf, rb0, rb1, accbuf,
              sem_v, sem_u, sem_n0, sem_n1, sem_w):
        wid = lax.axis_index("s") * _NC + lax.axis_index("c")
        base = wid * bpw

        pltpu.sync_copy(tgt_hbm.at[pl.ds(base, bpw)], tidx_v)
        pltpu.sync_copy(ctx_hbm.at[pl.ds(base, bpw)], cidx_v)
        pltpu.sync_copy(negw_hbm.at[wid], nidx_v)

        cp_v = pltpu.async_copy(iemb_hbm.at[tidx_v], vbuf, sem_v)
        cp_u = pltpu.async_copy(oemb_hbm.at[cidx_v], ubuf, sem_u)
        cp_a = pltpu.async_copy(oemb_hbm.at[nidx_v.at[0]], accbuf, sem_n0)

        rbs = (rb0, rb1)
        sems = (sem_n0, sem_n1)
        cps = {1: pltpu.async_copy(oemb_hbm.at[nidx_v.at[1]], rb1, sem_n1)}

        cp_v.wait()
        w_v = pltpu.async_copy(vbuf, v_out.at[pl.ds(base, bpw)], sem_w)
        cp_u.wait()
        w_u = pltpu.async_copy(ubuf, u_out.at[pl.ds(base, bpw)], sem_w)
        cp_a.wait()

        def accumulate(buf):
            def r_body(r, c2):
                for s in range(n_slices):
                    plsc.addupdate(
                        accbuf.at[r, pl.ds(s * _LANES, _LANES)],
                        buf[r, pl.ds(s * _LANES, _LANES)],
                    )
                return c2

            lax.fori_loop(0, bpw, r_body, 0)

        for n in range(1, NNEG):
            if n + 1 < NNEG:
                cps[n + 1] = pltpu.async_copy(
                    oemb_hbm.at[nidx_v.at[n + 1]], rbs[(n + 1) % 2],
                    sems[(n + 1) % 2])
            cps[n].wait()
            accumulate(rbs[n % 2])

        w_s = pltpu.async_copy(accbuf, us_out.at[pl.ds(base, bpw)], sem_w)
        w_v.wait()
        w_u.wait()
        w_s.wait()

    return sc_fn


def _tc_pad(V, D, DP, rows):
    assert V % rows == 0 and rows % 8 == 0

    def body(i_ref, o_ref):
        o_ref[:, :D] = i_ref[...]

    return pl.pallas_call(
        body,
        grid=(V // rows,),
        in_specs=[pl.BlockSpec((rows, D), lambda i: (i, 0))],
        out_specs=pl.BlockSpec((rows, DP), lambda i: (i, 0)),
        out_shape=jax.ShapeDtypeStruct((V, DP), jnp.float32),
    )


def _tc_tail(B, EMB, TNUM, D):
    def body(v_ref, u_ref, us_ref, ty_ref, tm_ref, wt_ref, loss_ref, tloss_ref):
        v = v_ref[...]
        tp = jnp.dot(v, wt_ref[...], preferred_element_type=jnp.float32)
        ty = ty_ref[...]
        tm = tm_ref[...]
        bce = tm * (jnp.maximum(tp, 0.0) - tp * ty
                    + jnp.log(1.0 + jnp.exp(-jnp.abs(tp))))
        tloss_ref[0, 0] = jnp.sum(bce) / (B * TNUM)

        sig = 1.0 / (1.0 + jnp.exp(-tp))
        u = u_ref[...]
        us = us_ref[...]
        pos = (jnp.sum(u[:, :EMB] * v, axis=1, keepdims=True)
               + jnp.sum(u[:, EMB:D] * sig, axis=1, keepdims=True))
        neg = (jnp.sum(us[:, :EMB] * v, axis=1, keepdims=True)
               + jnp.sum(us[:, EMB:D] * sig, axis=1, keepdims=True))

        def logsig(x):
            return jnp.minimum(x, 0.0) - jnp.log(1.0 + jnp.exp(-jnp.abs(x)))

        loss_vec = logsig(pos) + logsig(-neg)
        loss_ref[0, 0] = -(jnp.sum(loss_vec) / B)

    return pl.pallas_call(
        body,
        out_shape=(
            jax.ShapeDtypeStruct((1, 1), jnp.float32),
            jax.ShapeDtypeStruct((1, 1), jnp.float32),
        ),
        out_specs=(
            pl.BlockSpec(memory_space=pltpu.SMEM),
            pl.BlockSpec(memory_space=pltpu.SMEM),
        ),
    )


def kernel(target_input, type_input, context, types, neg, type_mask,
           input_emb, output_emb, type_W):
    del type_input
    B = target_input.shape[0]
    EMB = input_emb.shape[1]
    V, D = output_emb.shape
    TNUM = type_W.shape[0]
    NNEG = neg.shape[1]

    DP = (D + _LANES - 1) // _LANES * _LANES
    oemb_p = _tc_pad(V, D, DP, rows=5000)(output_emb)

    nw = _NC * _NS
    bpw = B // nw
    neg_w = jnp.transpose(neg.reshape(nw, bpw, NNEG), (0, 2, 1))
    sc = _sc_gather(B, V, EMB, DP, NNEG)
    v, u, usum = sc(target_input, context, neg_w, input_emb, oemb_p)

    loss, tloss = _tc_tail(B, EMB, TNUM, D)(
        v, u, usum, types, type_mask, jnp.transpose(type_W))
    return (loss[0, 0], tloss[0, 0])


# speedup vs baseline: 4.2120x; 1.0037x over previous
import functools

import jax
import jax.numpy as jnp
from jax import lax
from jax.experimental import pallas as pl
from jax.experimental.pallas import tpu as pltpu
from jax.experimental.pallas import tpu_sc as plsc

_NC = 2
_NS = 16
_LANES = 16


def _sc_gather(B, V, EMB, DP, NNEG):
    nw = _NC * _NS
    assert B % nw == 0
    bpw = B // nw
    assert bpw <= 128
    assert DP % _LANES == 0 and EMB % _LANES == 0
    n_slices = DP // _LANES

    mesh = plsc.VectorSubcoreMesh(core_axis_name="c", subcore_axis_name="s")

    @functools.partial(
        pl.kernel,
        out_type=(
            jax.ShapeDtypeStruct((B, EMB), jnp.float32),
            jax.ShapeDtypeStruct((B, DP), jnp.float32),
            jax.ShapeDtypeStruct((B, DP), jnp.float32),
        ),
        mesh=mesh,
        compiler_params=pltpu.CompilerParams(use_tc_tiling_on_sc=False),
        scratch_types=(
            pltpu.VMEM((bpw,), jnp.int32),
            pltpu.VMEM((bpw,), jnp.int32),
            pltpu.VMEM((NNEG, bpw), jnp.int32),
            pltpu.VMEM((bpw, EMB), jnp.float32),
            pltpu.VMEM((bpw, DP), jnp.float32),
            pltpu.VMEM((bpw, DP), jnp.float32),
            pltpu.VMEM((bpw, DP), jnp.float32),
            pltpu.VMEM((bpw, DP), jnp.float32),
            pltpu.SemaphoreType.DMA,
            pltpu.SemaphoreType.DMA,
            pltpu.SemaphoreType.DMA,
            pltpu.SemaphoreType.DMA,
            pltpu.SemaphoreType.DMA,
        ),
    )
    def sc_fn(tgt_hbm, ctx_hbm, negw_hbm, iemb_hbm, oemb_hbm,
              v_out, u_out, us_out,
              tidx_v, cidx_v, nidx_v, vbuf, ubuf, rb0, rb1, accbuf,
              sem_v, sem_u, sem_n0, sem_n1, sem_w):
        wid = lax.axis_index("s") * _NC + lax.axis_index("c")
        base = wid * bpw

        pltpu.sync_copy(tgt_hbm.at[pl.ds(base, bpw)], tidx_v)
        pltpu.sync_copy(ctx_hbm.at[pl.ds(base, bpw)], cidx_v)
        pltpu.sync_copy(negw_hbm.at[wid], nidx_v)

        cp_v = pltpu.async_copy(iemb_hbm.at[tidx_v], vbuf, sem_v)
        cp_u = pltpu.async_copy(oemb_hbm.at[cidx_v], ubuf, sem_u)
        cp_a = pltpu.async_copy(oemb_hbm.at[nidx_v.at[0]], accbuf, sem_n0)

        rbs = (rb0, rb1)
        sems = (sem_n0, sem_n1)
        cps = {1: pltpu.async_copy(oemb_hbm.at[nidx_v.at[1]], rb1, sem_n1)}

        cp_v.wait()
        w_v = pltpu.async_copy(vbuf, v_out.at[pl.ds(base, bpw)], sem_w)
        cp_u.wait()
        w_u = pltpu.async_copy(ubuf, u_out.at[pl.ds(base, bpw)], sem_w)
        cp_a.wait()

        def accumulate(buf):
            def r_body(r, c2):
                for s in range(n_slices):
                    plsc.addupdate(
                        accbuf.at[r, pl.ds(s * _LANES, _LANES)],
                        buf[r, pl.ds(s * _LANES, _LANES)],
                    )
                return c2

            lax.fori_loop(0, bpw, r_body, 0)

        for n in range(1, NNEG):
            if n + 1 < NNEG:
                cps[n + 1] = pltpu.async_copy(
                    oemb_hbm.at[nidx_v.at[n + 1]], rbs[(n + 1) % 2],
                    sems[(n + 1) % 2])
            cps[n].wait()
            accumulate(rbs[n % 2])

        w_s = pltpu.async_copy(accbuf, us_out.at[pl.ds(base, bpw)], sem_w)
        w_v.wait()
        w_u.wait()
        w_s.wait()

    return sc_fn


def _tc_pad(V, D, DP, rows):
    assert V % rows == 0 and rows % 8 == 0

    def body(i_ref, o_ref):
        o_ref[:, :D] = i_ref[...]

    return pl.pallas_call(
        body,
        grid=(V // rows,),
        in_specs=[pl.BlockSpec((rows, D), lambda i: (i, 0))],
        out_specs=pl.BlockSpec((rows, DP), lambda i: (i, 0)),
        out_shape=jax.ShapeDtypeStruct((V, DP), jnp.float32),
    )


def _tc_tail(B, EMB, TNUM, D):
    def body(v_ref, u_ref, us_ref, ty_ref, tm_ref, wt_ref, loss_ref, tloss_ref):
        v = v_ref[...]
        tp = jnp.dot(v, wt_ref[...], preferred_element_type=jnp.float32)
        ty = ty_ref[...]
        tm = tm_ref[...]
        bce = tm * (jnp.maximum(tp, 0.0) - tp * ty
                    + jnp.log(1.0 + jnp.exp(-jnp.abs(tp))))
        tloss_ref[0, 0] = jnp.sum(bce) / (B * TNUM)

        sig = 1.0 / (1.0 + jnp.exp(-tp))
        u = u_ref[...]
        us = us_ref[...]
        pos = (jnp.sum(u[:, :EMB] * v, axis=1, keepdims=True)
               + jnp.sum(u[:, EMB:D] * sig, axis=1, keepdims=True))
        neg = (jnp.sum(us[:, :EMB] * v, axis=1, keepdims=True)
               + jnp.sum(us[:, EMB:D] * sig, axis=1, keepdims=True))

        def logsig(x):
            return jnp.minimum(x, 0.0) - jnp.log(1.0 + jnp.exp(-jnp.abs(x)))

        loss_vec = logsig(pos) + logsig(-neg)
        loss_ref[0, 0] = -(jnp.sum(loss_vec) / B)

    return pl.pallas_call(
        body,
        out_shape=(
            jax.ShapeDtypeStruct((1, 1), jnp.float32),
            jax.ShapeDtypeStruct((1, 1), jnp.float32),
        ),
        out_specs=(
            pl.BlockSpec(memory_space=pltpu.SMEM),
            pl.BlockSpec(memory_space=pltpu.SMEM),
        ),
    )


def kernel(target_input, type_input, context, types, neg, type_mask,
           input_emb, output_emb, type_W):
    del type_input
    B = target_input.shape[0]
    EMB = input_emb.shape[1]
    V, D = output_emb.shape
    TNUM = type_W.shape[0]
    NNEG = neg.shape[1]

    DP = (D + _LANES - 1) // _LANES * _LANES
    oemb_p = _tc_pad(V, D, DP, rows=10000)(output_emb)

    nw = _NC * _NS
    bpw = B // nw
    neg_w = jnp.transpose(neg.reshape(nw, bpw, NNEG), (0, 2, 1))
    sc = _sc_gather(B, V, EMB, DP, NNEG)
    v, u, usum = sc(target_input, context, neg_w, input_emb, oemb_p)

    loss, tloss = _tc_tail(B, EMB, TNUM, D)(
        v, u, usum, types, type_mask, jnp.transpose(type_W))
    return (loss[0, 0], tloss[0, 0])
